# baseline (speedup 1.0000x reference)
import functools

import jax
import jax.numpy as jnp
from jax import lax
from jax.experimental import pallas as pl
from jax.experimental.pallas import tpu as pltpu
from jax.experimental.pallas import tpu_sc as plsc

_N = 10000
_E = 320000
_H = 8
_F = 16
_D1 = 144
_D2 = 16
_NC = 2
_NS = 16
_NTILES = _NC * _NS
_EPT = _E // _NTILES
_CHUNK = 50
_NCH = _EPT // _CHUNK
_CHUNK2 = 125
_RPS = 624
_RTAIL = _N - _NS * _RPS


def _tc1_body(x_ref, w_ref, al_ref, ar_ref, zext_ref, r_ref):
    z = jnp.dot(x_ref[...], w_ref[...], preferred_element_type=jnp.float32)
    el = jnp.dot(z, al_ref[...], preferred_element_type=jnp.float32)
    er = jnp.dot(z, ar_ref[...], preferred_element_type=jnp.float32)
    n = z.shape[0]
    zext_ref[...] = jnp.concatenate(
        [z, jnp.ones((n, _H), jnp.float32), el], axis=1)
    r_ref[...] = jnp.concatenate([er, er], axis=1)


def _tc1(x, W1, AL, AR):
    return pl.pallas_call(
        _tc1_body,
        out_shape=(
            jax.ShapeDtypeStruct((_N, _D1), jnp.float32),
            jax.ShapeDtypeStruct((_N, 16), jnp.float32),
        ),
    )(x, W1, AL, AR)


def _sc_edge_pass(src3, dst3, zext, rtab, d, multihead, chunk):
    nch = _EPT // chunk
    mesh = plsc.VectorSubcoreMesh(core_axis_name="c", subcore_axis_name="s")
    zeros = jnp.zeros((_N, d), jnp.float32)

    @functools.partial(
        pl.kernel,
        mesh=mesh,
        out_type=jax.ShapeDtypeStruct((_NC, _N, d), jnp.float32),
        compiler_params=pltpu.CompilerParams(use_tc_tiling_on_sc=False,
                                             needs_layout_passes=False),
        scratch_types=[
            pltpu.VMEM_SHARED((_N, d), jnp.float32),
            pltpu.VMEM((nch, chunk), jnp.int32),
            pltpu.VMEM((nch, chunk), jnp.int32),
            pltpu.VMEM((chunk, 16), jnp.float32),
            pltpu.VMEM((chunk, 16), jnp.float32),
            pltpu.VMEM((chunk, d), jnp.float32),
            pltpu.VMEM((chunk, d), jnp.float32),
            pltpu.SemaphoreType.DMA,
            pltpu.SemaphoreType.DMA,
            pltpu.SemaphoreType.DMA,
            pltpu.SemaphoreType.DMA,
        ],
    )
    def k(src_hbm, dst_hbm, z_hbm, r_hbm, zero_hbm, out_hbm,
          acc, srcv, dstv, rv0, rv1, zv0, zv1, sg0, sg1, ss0, ss1):
        cid = lax.axis_index("c")
        sid = lax.axis_index("s")
        wid = cid * _NS + sid
        rv = (rv0, rv1)
        zv = (zv0, zv1)
        sg = (sg0, sg1)
        ss = (ss0, ss1)

        pltpu.sync_copy(zero_hbm.at[pl.ds(sid * _RPS, _RPS)],
                        acc.at[pl.ds(sid * _RPS, _RPS)])

        @pl.when(sid == _NS - 1)
        def _zero_tail():
            pltpu.sync_copy(zero_hbm.at[pl.ds(_NS * _RPS, _RTAIL)],
                            acc.at[pl.ds(_NS * _RPS, _RTAIL)])

        pltpu.sync_copy(src_hbm.at[wid], srcv)
        pltpu.sync_copy(dst_hbm.at[wid], dstv)
        plsc.subcore_barrier()

        def issue_gather(b, g):
            pltpu.async_copy(r_hbm.at[dstv.at[g]], rv[b], sg[b])
            pltpu.async_copy(z_hbm.at[srcv.at[g]], zv[b], sg[b])

        def wait_gather(b, g):
            pltpu.make_async_copy(r_hbm.at[dstv.at[g]], rv[b], sg[b]).wait()
            pltpu.make_async_copy(z_hbm.at[srcv.at[g]], zv[b], sg[b]).wait()

        def issue_scatter(b, g):
            pltpu.async_copy(zv[b], acc.at[dstv.at[g]], ss[b], add=True)

        def wait_scatter(b, g):
            pltpu.make_async_copy(zv[b], acc.at[dstv.at[g]], ss[b]).wait()

        def compute(b):
            z = zv[b]
            r = rv[b]
            if multihead:
                @plsc.parallel_loop(0, chunk, unroll=5)
                def _msg(ci):
                    tail = z[ci, pl.ds(_H * _F, 16)]
                    e = tail + r[ci]
                    e = jnp.where(e >= 0.0, e, 0.2 * e)
                    w16 = jnp.exp(e)
                    for h in range(_H):
                        ws = w16[8 + h]
                        z[ci, pl.ds(h * _F, 16)] = ws * z[ci, pl.ds(h * _F, 16)]
                    z[ci, pl.ds(_H * _F, 16)] = lax.rev(w16, (0,)) * tail
            else:
                @plsc.parallel_loop(0, chunk, unroll=4)
                def _msg(ci):
                    zrow = z[ci]
                    bc = plsc.load_gather(
                        z, [jnp.full((16,), ci, jnp.int32),
                            jnp.full((16,), 4, jnp.int32)])
                    e = bc + r[ci]
                    e = jnp.where(e >= 0.0, e, 0.2 * e)
                    z[ci] = jnp.exp(e) * zrow

        issue_gather(0, 0)

        @pl.loop(0, nch, step=2)
        def _pair(g):
            issue_gather(1, g + 1)
            wait_gather(0, g)
            compute(0)
            issue_scatter(0, g)
            wait_gather(1, g + 1)
            compute(1)
            issue_scatter(1, g + 1)
            wait_scatter(0, g)

            @pl.when(g + 2 < nch)
            def _next():
                issue_gather(0, g + 2)

            wait_scatter(1, g + 1)

        plsc.subcore_barrier()
        pltpu.sync_copy(acc.at[pl.ds(sid * _RPS, _RPS)],
                        out_hbm.at[cid, pl.ds(sid * _RPS, _RPS)])

        @pl.when(sid == _NS - 1)
        def _out_tail():
            pltpu.sync_copy(acc.at[pl.ds(_NS * _RPS, _RTAIL)],
                            out_hbm.at[cid, pl.ds(_NS * _RPS, _RTAIL)])

    return k(src3, dst3, zext, rtab, zeros)


def _tc2_body(accp_ref, b1_ref, w2_ref, va_ref, vb_ref, expand_ref,
              zext_ref, r_ref):
    acc = accp_ref[0] + accp_ref[1]
    s = acc[:, _H * _F:_H * _F + _H]
    sx = jnp.dot(s, expand_ref[...], preferred_element_type=jnp.float32)
    h = acc[:, 0:_H * _F] / (sx + 1e-9) + b1_ref[...]
    h = jnp.where(h > 0.0, h, jnp.exp(h) - 1.0)
    z2 = jnp.dot(h, w2_ref[...], preferred_element_type=jnp.float32)
    el2 = jnp.dot(h, va_ref[...], preferred_element_type=jnp.float32)
    er2 = jnp.dot(h, vb_ref[...], preferred_element_type=jnp.float32)
    n = h.shape[0]
    one16 = jnp.ones((1, 16), jnp.float32)
    zext_ref[...] = jnp.concatenate(
        [z2, jnp.ones((n, 1), jnp.float32), el2,
         jnp.zeros((n, 11), jnp.float32)], axis=1)
    r_ref[...] = jnp.dot(er2, one16, preferred_element_type=jnp.float32)


def _tc2(accp, b1, W2, va, vb, EXPAND):
    return pl.pallas_call(
        _tc2_body,
        out_shape=(
            jax.ShapeDtypeStruct((_N, _D2), jnp.float32),
            jax.ShapeDtypeStruct((_N, 16), jnp.float32),
        ),
    )(accp, b1, W2, va, vb, EXPAND)


def _tc3_body(accp_ref, b2_ref, out_ref):
    acc = accp_ref[0] + accp_ref[1]
    sb = jnp.dot(acc[:, 3:4], jnp.ones((1, 16), jnp.float32),
                 preferred_element_type=jnp.float32)
    out_ref[...] = acc[:, 0:3] / (sb[:, 0:3] + 1e-9) + b2_ref[...]


def _tc3(accp, b2):
    return pl.pallas_call(
        _tc3_body,
        out_shape=jax.ShapeDtypeStruct((_N, 3), jnp.float32),
    )(accp, b2)


def kernel(features, edge_index, W1, attn_l1, attn_r1, b1,
           W2, attn_l2, attn_r2, b2):
    src3 = edge_index[0].reshape(_NTILES, _NCH, _CHUNK)
    dst3 = edge_index[1].reshape(_NTILES, _NCH, _CHUNK)
    eye8 = jnp.eye(_H, dtype=jnp.float32)
    AL = (attn_l1[:, :, None] * eye8[:, None, :]).reshape(_H * _F, _H)
    AR = (attn_r1[:, :, None] * eye8[:, None, :]).reshape(_H * _F, _H)
    EXPAND = jnp.kron(jnp.fliplr(eye8), jnp.ones((1, _F), jnp.float32))
    va = (W2 @ attn_l2[0]).reshape(_H * _F, 1)
    vb = (W2 @ attn_r2[0]).reshape(_H * _F, 1)

    zext1, R1 = _tc1(features, W1, AL, AR)
    acc1 = _sc_edge_pass(src3, dst3, zext1, R1, _D1, True, _CHUNK)
    src3b = edge_index[0].reshape(_NTILES, _EPT // _CHUNK2, _CHUNK2)
    dst3b = edge_index[1].reshape(_NTILES, _EPT // _CHUNK2, _CHUNK2)
    zext2, R2 = _tc2(acc1, b1.reshape(1, _H * _F), W2, va, vb, EXPAND)
    acc2 = _sc_edge_pass(src3b, dst3b, zext2, R2, _D2, False, _CHUNK2)
    out = _tc3(acc2, b2.reshape(1, 3))
    return out.reshape(_N, 1, 3)


# speedup vs baseline: 1.2820x; 1.2820x over previous
import functools

import jax
import jax.numpy as jnp
from jax import lax
from jax.experimental import pallas as pl
from jax.experimental.pallas import tpu as pltpu
from jax.experimental.pallas import tpu_sc as plsc

_N = 10000
_E = 320000
_H = 8
_F = 16
_D1 = 144
_D2 = 16
_NC = 2
_NS = 16
_NTILES = _NC * _NS
_EPT = _E // _NTILES
_CHUNK = 50
_NCH = _EPT // _CHUNK
_CHUNK2 = 125
_RPS = 624
_RTAIL = _N - _NS * _RPS


def _tc1_body(x_ref, w_ref, al_ref, ar_ref, zext_ref, r_ref):
    z = jnp.dot(x_ref[...], w_ref[...], preferred_element_type=jnp.float32)
    el = jnp.dot(z, al_ref[...], preferred_element_type=jnp.float32)
    er = jnp.dot(z, ar_ref[...], preferred_element_type=jnp.float32)
    n = z.shape[0]
    zext_ref[...] = jnp.concatenate(
        [z, jnp.ones((n, _H), jnp.float32), el], axis=1)
    r_ref[...] = jnp.concatenate([er, er], axis=1)


def _tc1(x, W1, AL, AR):
    return pl.pallas_call(
        _tc1_body,
        out_shape=(
            jax.ShapeDtypeStruct((_N, _D1), jnp.float32),
            jax.ShapeDtypeStruct((_N, 16), jnp.float32),
        ),
    )(x, W1, AL, AR)


def _sc_edge_pass(src3, dst3, zext, rtab, d, multihead, chunk):
    nch = _EPT // chunk
    mesh = plsc.VectorSubcoreMesh(core_axis_name="c", subcore_axis_name="s")
    zeros = jnp.zeros((_N, d), jnp.float32)

    @functools.partial(
        pl.kernel,
        mesh=mesh,
        out_type=jax.ShapeDtypeStruct((_NC, _N, d), jnp.float32),
        compiler_params=pltpu.CompilerParams(use_tc_tiling_on_sc=False,
                                             needs_layout_passes=False),
        scratch_types=[
            pltpu.VMEM_SHARED((_N, d), jnp.float32),
            pltpu.VMEM((nch, chunk), jnp.int32),
            pltpu.VMEM((nch, chunk), jnp.int32),
            pltpu.VMEM((chunk, 16), jnp.float32),
            pltpu.VMEM((chunk, 16), jnp.float32),
            pltpu.VMEM((chunk, d), jnp.float32),
            pltpu.VMEM((chunk, d), jnp.float32),
            pltpu.SemaphoreType.DMA,
            pltpu.SemaphoreType.DMA,
            pltpu.SemaphoreType.DMA,
            pltpu.SemaphoreType.DMA,
        ],
    )
    def k(src_hbm, dst_hbm, z_hbm, r_hbm, zero_hbm, out_hbm,
          acc, srcv, dstv, rv0, rv1, zv0, zv1, sg0, sg1, ss0, ss1):
        cid = lax.axis_index("c")
        sid = lax.axis_index("s")
        wid = cid * _NS + sid
        rv = (rv0, rv1)
        zv = (zv0, zv1)
        sg = (sg0, sg1)
        ss = (ss0, ss1)

        pltpu.sync_copy(zero_hbm.at[pl.ds(sid * _RPS, _RPS)],
                        acc.at[pl.ds(sid * _RPS, _RPS)])

        @pl.when(sid == _NS - 1)
        def _zero_tail():
            pltpu.sync_copy(zero_hbm.at[pl.ds(_NS * _RPS, _RTAIL)],
                            acc.at[pl.ds(_NS * _RPS, _RTAIL)])

        pltpu.sync_copy(src_hbm.at[wid], srcv)
        pltpu.sync_copy(dst_hbm.at[wid], dstv)
        plsc.subcore_barrier()

        def issue_gather(b, g):
            pltpu.async_copy(r_hbm.at[dstv.at[g]], rv[b], sg[b])
            pltpu.async_copy(z_hbm.at[srcv.at[g]], zv[b], sg[b])

        def wait_gather(b, g):
            pltpu.make_async_copy(r_hbm.at[dstv.at[g]], rv[b], sg[b]).wait()
            pltpu.make_async_copy(z_hbm.at[srcv.at[g]], zv[b], sg[b]).wait()

        def issue_scatter(b, g):
            pltpu.async_copy(zv[b], acc.at[dstv.at[g]], ss[b], add=True)

        def wait_scatter(b, g):
            pltpu.make_async_copy(zv[b], acc.at[dstv.at[g]], ss[b]).wait()

        def compute(b):
            z = zv[b]
            r = rv[b]
            if multihead:
                @plsc.parallel_loop(0, chunk, unroll=3)
                def _msg(ci):
                    tail = z[ci, pl.ds(_H * _F, 16)]
                    e = tail + r[ci]
                    e = jnp.where(e >= 0.0, e, 0.2 * e)
                    w16 = jnp.exp(e)
                    for h in range(_H):
                        ws = w16[8 + h]
                        z[ci, pl.ds(h * _F, 16)] = ws * z[ci, pl.ds(h * _F, 16)]
                    z[ci, pl.ds(_H * _F, 16)] = lax.rev(w16, (0,)) * tail
            else:
                @plsc.parallel_loop(0, chunk, unroll=4)
                def _msg(ci):
                    zrow = z[ci]
                    bc = plsc.load_gather(
                        z, [jnp.full((16,), ci, jnp.int32),
                            jnp.full((16,), 4, jnp.int32)])
                    e = bc + r[ci]
                    e = jnp.where(e >= 0.0, e, 0.2 * e)
                    z[ci] = jnp.exp(e) * zrow

        issue_gather(0, 0)

        @pl.loop(0, nch, step=2)
        def _pair(g):
            issue_gather(1, g + 1)
            wait_gather(0, g)
            compute(0)
            issue_scatter(0, g)
            wait_gather(1, g + 1)
            compute(1)
            issue_scatter(1, g + 1)
            wait_scatter(0, g)

            @pl.when(g + 2 < nch)
            def _next():
                issue_gather(0, g + 2)

            wait_scatter(1, g + 1)

        plsc.subcore_barrier()
        pltpu.sync_copy(acc.at[pl.ds(sid * _RPS, _RPS)],
                        out_hbm.at[cid, pl.ds(sid * _RPS, _RPS)])

        @pl.when(sid == _NS - 1)
        def _out_tail():
            pltpu.sync_copy(acc.at[pl.ds(_NS * _RPS, _RTAIL)],
                            out_hbm.at[cid, pl.ds(_NS * _RPS, _RTAIL)])

    return k(src3, dst3, zext, rtab, zeros)


def _tc2_body(accp_ref, b1_ref, w2_ref, va_ref, vb_ref, expand_ref,
              zext_ref, r_ref):
    acc = accp_ref[0] + accp_ref[1]
    s = acc[:, _H * _F:_H * _F + _H]
    sx = jnp.dot(s, expand_ref[...], preferred_element_type=jnp.float32)
    h = acc[:, 0:_H * _F] / (sx + 1e-9) + b1_ref[...]
    h = jnp.where(h > 0.0, h, jnp.exp(h) - 1.0)
    z2 = jnp.dot(h, w2_ref[...], preferred_element_type=jnp.float32)
    el2 = jnp.dot(h, va_ref[...], preferred_element_type=jnp.float32)
    er2 = jnp.dot(h, vb_ref[...], preferred_element_type=jnp.float32)
    n = h.shape[0]
    one16 = jnp.ones((1, 16), jnp.float32)
    zext_ref[...] = jnp.concatenate(
        [z2, jnp.ones((n, 1), jnp.float32), el2,
         jnp.zeros((n, 11), jnp.float32)], axis=1)
    r_ref[...] = jnp.dot(er2, one16, preferred_element_type=jnp.float32)


def _tc2(accp, b1, W2, va, vb, EXPAND):
    return pl.pallas_call(
        _tc2_body,
        out_shape=(
            jax.ShapeDtypeStruct((_N, _D2), jnp.float32),
            jax.ShapeDtypeStruct((_N, 16), jnp.float32),
        ),
    )(accp, b1, W2, va, vb, EXPAND)


def _tc3_body(accp_ref, b2_ref, out_ref):
    acc = accp_ref[0] + accp_ref[1]
    sb = jnp.dot(acc[:, 3:4], jnp.ones((1, 16), jnp.float32),
                 preferred_element_type=jnp.float32)
    out_ref[...] = acc[:, 0:3] / (sb[:, 0:3] + 1e-9) + b2_ref[...]


def _tc3(accp, b2):
    return pl.pallas_call(
        _tc3_body,
        out_shape=jax.ShapeDtypeStruct((_N, 3), jnp.float32),
    )(accp, b2)


def kernel(features, edge_index, W1, attn_l1, attn_r1, b1,
           W2, attn_l2, attn_r2, b2):
    src3 = edge_index[0].reshape(_NTILES, _NCH, _CHUNK)
    dst3 = edge_index[1].reshape(_NTILES, _NCH, _CHUNK)
    eye8 = jnp.eye(_H, dtype=jnp.float32)
    AL = (attn_l1[:, :, None] * eye8[:, None, :]).reshape(_H * _F, _H)
    AR = (attn_r1[:, :, None] * eye8[:, None, :]).reshape(_H * _F, _H)
    EXPAND = jnp.kron(jnp.fliplr(eye8), jnp.ones((1, _F), jnp.float32))
    va = (W2 @ attn_l2[0]).reshape(_H * _F, 1)
    vb = (W2 @ attn_r2[0]).reshape(_H * _F, 1)

    zext1, R1 = _tc1(features, W1, AL, AR)
    acc1 = _sc_edge_pass(src3, dst3, zext1, R1, _D1, True, _CHUNK)
    src3b = edge_index[0].reshape(_NTILES, _EPT // _CHUNK2, _CHUNK2)
    dst3b = edge_index[1].reshape(_NTILES, _EPT // _CHUNK2, _CHUNK2)
    zext2, R2 = _tc2(acc1, b1.reshape(1, _H * _F), W2, va, vb, EXPAND)
    acc2 = _sc_edge_pass(src3b, dst3b, zext2, R2, _D2, False, _CHUNK2)
    out = _tc3(acc2, b2.reshape(1, 3))
    return out.reshape(_N, 1, 3)


# speedup vs baseline: 1.4136x; 1.1027x over previous
import functools

import jax
import jax.numpy as jnp
from jax import lax
from jax.experimental import pallas as pl
from jax.experimental.pallas import tpu as pltpu
from jax.experimental.pallas import tpu_sc as plsc

_N = 10000
_E = 320000
_H = 8
_F = 16
_D1 = 144
_D2 = 16
_NC = 2
_NS = 16
_NTILES = _NC * _NS
_EPT = _E // _NTILES
_CHUNK = 50
_NCH = _EPT // _CHUNK
_CHUNK2 = 125
_RPS = 624
_RTAIL = _N - _NS * _RPS


def _tc1_body(x_ref, w_ref, al_ref, ar_ref, zext_ref, r_ref):
    z = jnp.dot(x_ref[...], w_ref[...], preferred_element_type=jnp.float32)
    el = jnp.dot(z, al_ref[...], preferred_element_type=jnp.float32)
    er = jnp.dot(z, ar_ref[...], preferred_element_type=jnp.float32)
    n = z.shape[0]
    zext_ref[...] = jnp.concatenate(
        [z, jnp.ones((n, _H), jnp.float32), el], axis=1)
    r_ref[...] = jnp.concatenate([er, er], axis=1)


def _tc1(x, W1, AL, AR):
    return pl.pallas_call(
        _tc1_body,
        out_shape=(
            jax.ShapeDtypeStruct((_N, _D1), jnp.float32),
            jax.ShapeDtypeStruct((_N, 16), jnp.float32),
        ),
    )(x, W1, AL, AR)


def _sc_edge_pass(src3, dst3, zext, rtab, d, multihead, chunk):
    nch = _EPT // chunk
    mesh = plsc.VectorSubcoreMesh(core_axis_name="c", subcore_axis_name="s")
    zeros = jnp.zeros((_N, d), jnp.float32)

    @functools.partial(
        pl.kernel,
        mesh=mesh,
        out_type=jax.ShapeDtypeStruct((_NC, _N, d), jnp.float32),
        compiler_params=pltpu.CompilerParams(use_tc_tiling_on_sc=False,
                                             needs_layout_passes=False),
        scratch_types=[
            pltpu.VMEM_SHARED((_N, d), jnp.float32),
            pltpu.VMEM((nch, chunk), jnp.int32),
            pltpu.VMEM((nch, chunk), jnp.int32),
            pltpu.VMEM((chunk, 16), jnp.float32),
            pltpu.VMEM((chunk, 16), jnp.float32),
            pltpu.VMEM((chunk, d), jnp.float32),
            pltpu.VMEM((chunk, d), jnp.float32),
            pltpu.SemaphoreType.DMA,
            pltpu.SemaphoreType.DMA,
            pltpu.SemaphoreType.DMA,
            pltpu.SemaphoreType.DMA,
        ],
    )
    def k(src_hbm, dst_hbm, z_hbm, r_hbm, zero_hbm, out_hbm,
          acc, srcv, dstv, rv0, rv1, zv0, zv1, sg0, sg1, ss0, ss1):
        cid = lax.axis_index("c")
        sid = lax.axis_index("s")
        wid = cid * _NS + sid
        rv = (rv0, rv1)
        zv = (zv0, zv1)
        sg = (sg0, sg1)
        ss = (ss0, ss1)

        pltpu.sync_copy(zero_hbm.at[pl.ds(sid * _RPS, _RPS)],
                        acc.at[pl.ds(sid * _RPS, _RPS)])

        @pl.when(sid == _NS - 1)
        def _zero_tail():
            pltpu.sync_copy(zero_hbm.at[pl.ds(_NS * _RPS, _RTAIL)],
                            acc.at[pl.ds(_NS * _RPS, _RTAIL)])

        pltpu.sync_copy(src_hbm.at[wid], srcv)
        pltpu.sync_copy(dst_hbm.at[wid], dstv)
        plsc.subcore_barrier()

        def issue_gather(b, g):
            pltpu.async_copy(r_hbm.at[dstv.at[g]], rv[b], sg[b])
            pltpu.async_copy(z_hbm.at[srcv.at[g]], zv[b], sg[b])

        def wait_gather(b, g):
            pltpu.make_async_copy(r_hbm.at[dstv.at[g]], rv[b], sg[b]).wait()
            pltpu.make_async_copy(z_hbm.at[srcv.at[g]], zv[b], sg[b]).wait()

        def issue_scatter(b, g):
            pltpu.async_copy(zv[b], acc.at[dstv.at[g]], ss[b], add=True)

        def wait_scatter(b, g):
            pltpu.make_async_copy(zv[b], acc.at[dstv.at[g]], ss[b]).wait()

        def compute(b):
            z = zv[b]
            r = rv[b]
            if multihead:
                @plsc.parallel_loop(0, chunk, unroll=2)
                def _msg(ci):
                    tail = z[ci, pl.ds(_H * _F, 16)]
                    e = tail + r[ci]
                    e = jnp.where(e >= 0.0, e, 0.2 * e)
                    w16 = jnp.exp(e)
                    for h in range(_H):
                        ws = w16[8 + h]
                        z[ci, pl.ds(h * _F, 16)] = ws * z[ci, pl.ds(h * _F, 16)]
                    z[ci, pl.ds(_H * _F, 16)] = lax.rev(w16, (0,)) * tail
            else:
                @plsc.parallel_loop(0, chunk, unroll=4)
                def _msg(ci):
                    zrow = z[ci]
                    bc = plsc.load_gather(
                        z, [jnp.full((16,), ci, jnp.int32),
                            jnp.full((16,), 4, jnp.int32)])
                    e = bc + r[ci]
                    e = jnp.where(e >= 0.0, e, 0.2 * e)
                    z[ci] = jnp.exp(e) * zrow

        issue_gather(0, 0)

        @pl.loop(0, nch, step=2)
        def _pair(g):
            issue_gather(1, g + 1)
            wait_gather(0, g)
            compute(0)
            issue_scatter(0, g)
            wait_gather(1, g + 1)
            compute(1)
            issue_scatter(1, g + 1)
            wait_scatter(0, g)

            @pl.when(g + 2 < nch)
            def _next():
                issue_gather(0, g + 2)

            wait_scatter(1, g + 1)

        plsc.subcore_barrier()
        pltpu.sync_copy(acc.at[pl.ds(sid * _RPS, _RPS)],
                        out_hbm.at[cid, pl.ds(sid * _RPS, _RPS)])

        @pl.when(sid == _NS - 1)
        def _out_tail():
            pltpu.sync_copy(acc.at[pl.ds(_NS * _RPS, _RTAIL)],
                            out_hbm.at[cid, pl.ds(_NS * _RPS, _RTAIL)])

    return k(src3, dst3, zext, rtab, zeros)


_C2 = 80
_NCH2 = _EPT // _C2


def _sc_edge_pass2(src3, dst3, tab):
    mesh = plsc.VectorSubcoreMesh(core_axis_name="c", subcore_axis_name="s")
    zeros = jnp.zeros((_N, _D2), jnp.float32)

    @functools.partial(
        pl.kernel,
        mesh=mesh,
        out_type=jax.ShapeDtypeStruct((_NC, _N, _D2), jnp.float32),
        compiler_params=pltpu.CompilerParams(use_tc_tiling_on_sc=False,
                                             needs_layout_passes=False),
        scratch_types=[
            pltpu.VMEM_SHARED((_N, _D2), jnp.float32),
            pltpu.VMEM((_N, 8), jnp.float32),
            pltpu.VMEM((_NCH2, _C2), jnp.int32),
            pltpu.VMEM((_NCH2, _C2), jnp.int32),
            pltpu.VMEM((_C2, _D2), jnp.float32),
            pltpu.VMEM((_C2, _D2), jnp.float32),
            pltpu.SemaphoreType.DMA,
            pltpu.SemaphoreType.DMA,
        ],
    )
    def k(src_hbm, dst_hbm, tab_hbm, zero_hbm, out_hbm,
          acc, tabv, srcv, dstv, zv0, zv1, ss0, ss1):
        cid = lax.axis_index("c")
        sid = lax.axis_index("s")
        wid = cid * _NS + sid
        zv = (zv0, zv1)
        ss = (ss0, ss1)

        pltpu.sync_copy(zero_hbm.at[pl.ds(sid * _RPS, _RPS)],
                        acc.at[pl.ds(sid * _RPS, _RPS)])

        @pl.when(sid == _NS - 1)
        def _zero_tail():
            pltpu.sync_copy(zero_hbm.at[pl.ds(_NS * _RPS, _RTAIL)],
                            acc.at[pl.ds(_NS * _RPS, _RTAIL)])

        pltpu.sync_copy(tab_hbm, tabv)
        pltpu.sync_copy(src_hbm.at[wid], srcv)
        pltpu.sync_copy(dst_hbm.at[wid], dstv)
        plsc.subcore_barrier()

        iota16 = lax.iota(jnp.int32, 16)

        def compute(b, g):
            z = zv[b]

            @plsc.parallel_loop(0, _C2 // 16)
            def _grp(j):
                sv = srcv[g, pl.ds(j * 16, 16)]
                dv = dstv[g, pl.ds(j * 16, 16)]
                el = plsc.load_gather(tabv, [sv, jnp.full((16,), 0, jnp.int32)])
                er = plsc.load_gather(tabv, [dv, jnp.full((16,), 1, jnp.int32)])
                e = el + er
                e = jnp.where(e >= 0.0, e, 0.2 * e)
                w = jnp.exp(e)
                rows = j * 16 + iota16
                for c in range(3):
                    zc = plsc.load_gather(
                        tabv, [sv, jnp.full((16,), 2 + c, jnp.int32)])
                    plsc.store_scatter(
                        z, [rows, jnp.full((16,), c, jnp.int32)], w * zc)
                plsc.store_scatter(
                    z, [rows, jnp.full((16,), 3, jnp.int32)], w)

        def issue_scatter(b, g):
            pltpu.async_copy(zv[b], acc.at[dstv.at[g]], ss[b], add=True)

        def wait_scatter(b, g):
            pltpu.make_async_copy(zv[b], acc.at[dstv.at[g]], ss[b]).wait()

        @pl.loop(0, _NCH2 - 1, step=2)
        def _pair(g):
            compute(0, g)
            issue_scatter(0, g)
            compute(1, g + 1)
            issue_scatter(1, g + 1)
            wait_scatter(0, g)
            wait_scatter(1, g + 1)

        compute(0, _NCH2 - 1)
        issue_scatter(0, _NCH2 - 1)
        wait_scatter(0, _NCH2 - 1)

        plsc.subcore_barrier()
        pltpu.sync_copy(acc.at[pl.ds(sid * _RPS, _RPS)],
                        out_hbm.at[cid, pl.ds(sid * _RPS, _RPS)])

        @pl.when(sid == _NS - 1)
        def _out_tail():
            pltpu.sync_copy(acc.at[pl.ds(_NS * _RPS, _RTAIL)],
                            out_hbm.at[cid, pl.ds(_NS * _RPS, _RTAIL)])

    return k(src3, dst3, tab, zeros)


def _tc2_body(accp_ref, b1_ref, w2_ref, va_ref, vb_ref, expand_ref, tab_ref):
    acc = accp_ref[0] + accp_ref[1]
    s = acc[:, _H * _F:_H * _F + _H]
    sx = jnp.dot(s, expand_ref[...], preferred_element_type=jnp.float32)
    h = acc[:, 0:_H * _F] / (sx + 1e-9) + b1_ref[...]
    h = jnp.where(h > 0.0, h, jnp.exp(h) - 1.0)
    z2 = jnp.dot(h, w2_ref[...], preferred_element_type=jnp.float32)
    el2 = jnp.dot(h, va_ref[...], preferred_element_type=jnp.float32)
    er2 = jnp.dot(h, vb_ref[...], preferred_element_type=jnp.float32)
    n = h.shape[0]
    tab_ref[...] = jnp.concatenate(
        [el2, er2, z2, jnp.zeros((n, 3), jnp.float32)], axis=1)


def _tc2(accp, b1, W2, va, vb, EXPAND):
    return pl.pallas_call(
        _tc2_body,
        out_shape=jax.ShapeDtypeStruct((_N, 8), jnp.float32),
    )(accp, b1, W2, va, vb, EXPAND)


def _tc3_body(accp_ref, b2_ref, out_ref):
    acc = accp_ref[0] + accp_ref[1]
    sb = jnp.dot(acc[:, 3:4], jnp.ones((1, 16), jnp.float32),
                 preferred_element_type=jnp.float32)
    out_ref[...] = acc[:, 0:3] / (sb[:, 0:3] + 1e-9) + b2_ref[...]


def _tc3(accp, b2):
    return pl.pallas_call(
        _tc3_body,
        out_shape=jax.ShapeDtypeStruct((_N, 3), jnp.float32),
    )(accp, b2)


def kernel(features, edge_index, W1, attn_l1, attn_r1, b1,
           W2, attn_l2, attn_r2, b2):
    src3 = edge_index[0].reshape(_NTILES, _NCH, _CHUNK)
    dst3 = edge_index[1].reshape(_NTILES, _NCH, _CHUNK)
    eye8 = jnp.eye(_H, dtype=jnp.float32)
    AL = (attn_l1[:, :, None] * eye8[:, None, :]).reshape(_H * _F, _H)
    AR = (attn_r1[:, :, None] * eye8[:, None, :]).reshape(_H * _F, _H)
    EXPAND = jnp.kron(jnp.fliplr(eye8), jnp.ones((1, _F), jnp.float32))
    va = (W2 @ attn_l2[0]).reshape(_H * _F, 1)
    vb = (W2 @ attn_r2[0]).reshape(_H * _F, 1)

    zext1, R1 = _tc1(features, W1, AL, AR)
    acc1 = _sc_edge_pass(src3, dst3, zext1, R1, _D1, True, _CHUNK)
    src3b = edge_index[0].reshape(_NTILES, _NCH2, _C2)
    dst3b = edge_index[1].reshape(_NTILES, _NCH2, _C2)
    tab2 = _tc2(acc1, b1.reshape(1, _H * _F), W2, va, vb, EXPAND)
    acc2 = _sc_edge_pass2(src3b, dst3b, tab2)
    out = _tc3(acc2, b2.reshape(1, 3))
    return out.reshape(_N, 1, 3)


# speedup vs baseline: 1.5472x; 1.0945x over previous
import functools

import jax
import jax.numpy as jnp
from jax import lax
from jax.experimental import pallas as pl
from jax.experimental.pallas import tpu as pltpu
from jax.experimental.pallas import tpu_sc as plsc

_N = 10000
_E = 320000
_H = 8
_F = 16
_D1 = 136
_D2 = 16
_NC = 2
_NS = 16
_NTILES = _NC * _NS
_EPT = _E // _NTILES
_CHUNK = 100
_NCH = _EPT // _CHUNK
_RPS = 624
_RTAIL = _N - _NS * _RPS


def _tc1_body(x_ref, w_ref, al_ref, ar_ref, zext_ref, r_ref):
    z = jnp.dot(x_ref[...], w_ref[...], preferred_element_type=jnp.float32)
    el = jnp.dot(z, al_ref[...], preferred_element_type=jnp.float32)
    er = jnp.dot(z, ar_ref[...], preferred_element_type=jnp.float32)
    zext_ref[...] = jnp.concatenate([z, el], axis=1)
    r_ref[...] = jnp.concatenate([er, er], axis=1)


def _tc1(x, W1, AL, AR):
    return pl.pallas_call(
        _tc1_body,
        out_shape=(
            jax.ShapeDtypeStruct((_N, _D1), jnp.float32),
            jax.ShapeDtypeStruct((_N, 16), jnp.float32),
        ),
    )(x, W1, AL, AR)


def _sc_edge_pass1(src3, dst3, zext, rtab):
    mesh = plsc.VectorSubcoreMesh(core_axis_name="c", subcore_axis_name="s")
    zeros = jnp.zeros((_N, _D1), jnp.float32)

    @functools.partial(
        pl.kernel,
        mesh=mesh,
        out_type=jax.ShapeDtypeStruct((_NC, _N, _D1), jnp.float32),
        compiler_params=pltpu.CompilerParams(use_tc_tiling_on_sc=False,
                                             needs_layout_passes=False),
        scratch_types=[
            pltpu.VMEM_SHARED((_N, _D1), jnp.float32),
            pltpu.VMEM((_NCH, _CHUNK), jnp.int32),
            pltpu.VMEM((_CHUNK,), jnp.int32),
            pltpu.VMEM((_CHUNK,), jnp.int32),
            pltpu.VMEM((_CHUNK, 16), jnp.float32),
            pltpu.VMEM((_CHUNK, 16), jnp.float32),
            pltpu.VMEM((_CHUNK, _D1), jnp.float32),
            pltpu.VMEM((_CHUNK, _D1), jnp.float32),
            pltpu.SemaphoreType.DMA,
            pltpu.SemaphoreType.DMA,
            pltpu.SemaphoreType.DMA,
            pltpu.SemaphoreType.DMA,
            pltpu.SemaphoreType.DMA,
            pltpu.SemaphoreType.DMA,
        ],
    )
    def k(src_hbm, dst_hbm, z_hbm, r_hbm, zero_hbm, out_hbm,
          acc, dstv, sv0, sv1, rv0, rv1, zv0, zv1,
          sg0, sg1, ss0, ss1, si0, si1):
        cid = lax.axis_index("c")
        sid = lax.axis_index("s")
        wid = cid * _NS + sid
        sv = (sv0, sv1)
        rv = (rv0, rv1)
        zv = (zv0, zv1)
        sg = (sg0, sg1)
        ss = (ss0, ss1)
        si = (si0, si1)

        pltpu.sync_copy(zero_hbm.at[pl.ds(sid * _RPS, _RPS)],
                        acc.at[pl.ds(sid * _RPS, _RPS)])

        @pl.when(sid == _NS - 1)
        def _zero_tail():
            pltpu.sync_copy(zero_hbm.at[pl.ds(_NS * _RPS, _RTAIL)],
                            acc.at[pl.ds(_NS * _RPS, _RTAIL)])

        pltpu.sync_copy(dst_hbm.at[wid], dstv)
        plsc.subcore_barrier()

        iota16 = lax.iota(jnp.int32, 16)

        def issue_src(b, g):
            pltpu.async_copy(src_hbm.at[wid, g], sv[b], si[b])

        def wait_src(b, g):
            pltpu.make_async_copy(src_hbm.at[wid, g], sv[b], si[b]).wait()

        def issue_gather(b, g):
            pltpu.async_copy(r_hbm.at[dstv.at[g]], rv[b], sg[b])
            pltpu.async_copy(z_hbm.at[sv[b]], zv[b], sg[b])

        def wait_gather(b, g):
            pltpu.make_async_copy(r_hbm.at[dstv.at[g]], rv[b], sg[b]).wait()
            pltpu.make_async_copy(z_hbm.at[sv[b]], zv[b], sg[b]).wait()

        def issue_scatter(b, g):
            pltpu.async_copy(zv[b], acc.at[dstv.at[g]], ss[b], add=True)

        def wait_scatter(b, g):
            pltpu.make_async_copy(zv[b], acc.at[dstv.at[g]], ss[b]).wait()

        def compute(b):
            z = zv[b]
            r = rv[b]

            @plsc.parallel_loop(0, _CHUNK, unroll=2)
            def _msg(ci):
                v = z[ci, pl.ds(120, 16)]
                e = v + r[ci]
                e = jnp.where(e >= 0.0, e, 0.2 * e)
                w16 = jnp.exp(e)
                for h in range(_H):
                    ws = w16[8 + h]
                    z[ci, pl.ds(h * _F, 16)] = ws * z[ci, pl.ds(h * _F, 16)]
                cur = z[ci, pl.ds(120, 16)]
                z[ci, pl.ds(120, 16)] = jnp.where(iota16 < 8, cur, w16)

        pltpu.sync_copy(src_hbm.at[wid, 0], sv0)
        issue_gather(0, 0)
        issue_src(1, 1)

        @pl.loop(0, _NCH, step=2)
        def _pair(g):
            wait_src(1, g + 1)
            issue_gather(1, g + 1)
            wait_gather(0, g)

            @pl.when(g + 2 < _NCH)
            def _s0():
                issue_src(0, g + 2)

            compute(0)
            issue_scatter(0, g)
            wait_gather(1, g + 1)

            @pl.when(g + 3 < _NCH)
            def _s1():
                issue_src(1, g + 3)

            compute(1)
            issue_scatter(1, g + 1)
            wait_scatter(0, g)

            @pl.when(g + 2 < _NCH)
            def _g0():
                wait_src(0, g + 2)
                issue_gather(0, g + 2)

            wait_scatter(1, g + 1)

        plsc.subcore_barrier()
        pltpu.sync_copy(acc.at[pl.ds(sid * _RPS, _RPS)],
                        out_hbm.at[cid, pl.ds(sid * _RPS, _RPS)])

        @pl.when(sid == _NS - 1)
        def _out_tail():
            pltpu.sync_copy(acc.at[pl.ds(_NS * _RPS, _RTAIL)],
                            out_hbm.at[cid, pl.ds(_NS * _RPS, _RTAIL)])

    return k(src3, dst3, zext, rtab, zeros)


_C2 = 80
_NCH2 = _EPT // _C2


def _sc_edge_pass2(src3, dst3, tab):
    mesh = plsc.VectorSubcoreMesh(core_axis_name="c", subcore_axis_name="s")
    zeros = jnp.zeros((_N, _D2), jnp.float32)

    @functools.partial(
        pl.kernel,
        mesh=mesh,
        out_type=jax.ShapeDtypeStruct((_NC, _N, _D2), jnp.float32),
        compiler_params=pltpu.CompilerParams(use_tc_tiling_on_sc=False,
                                             needs_layout_passes=False),
        scratch_types=[
            pltpu.VMEM_SHARED((_N, _D2), jnp.float32),
            pltpu.VMEM((_N, 8), jnp.float32),
            pltpu.VMEM((_NCH2, _C2), jnp.int32),
            pltpu.VMEM((_NCH2, _C2), jnp.int32),
            pltpu.VMEM((_C2, _D2), jnp.float32),
            pltpu.VMEM((_C2, _D2), jnp.float32),
            pltpu.SemaphoreType.DMA,
            pltpu.SemaphoreType.DMA,
        ],
    )
    def k(src_hbm, dst_hbm, tab_hbm, zero_hbm, out_hbm,
          acc, tabv, srcv, dstv, zv0, zv1, ss0, ss1):
        cid = lax.axis_index("c")
        sid = lax.axis_index("s")
        wid = cid * _NS + sid
        zv = (zv0, zv1)
        ss = (ss0, ss1)

        pltpu.sync_copy(zero_hbm.at[pl.ds(sid * _RPS, _RPS)],
                        acc.at[pl.ds(sid * _RPS, _RPS)])

        @pl.when(sid == _NS - 1)
        def _zero_tail():
            pltpu.sync_copy(zero_hbm.at[pl.ds(_NS * _RPS, _RTAIL)],
                            acc.at[pl.ds(_NS * _RPS, _RTAIL)])

        pltpu.sync_copy(tab_hbm, tabv)
        pltpu.sync_copy(src_hbm.at[wid], srcv)
        pltpu.sync_copy(dst_hbm.at[wid], dstv)
        plsc.subcore_barrier()

        iota16 = lax.iota(jnp.int32, 16)

        def compute(b, g):
            z = zv[b]

            @plsc.parallel_loop(0, _C2 // 16)
            def _grp(j):
                sv = srcv[g, pl.ds(j * 16, 16)]
                dv = dstv[g, pl.ds(j * 16, 16)]
                el = plsc.load_gather(tabv, [sv, jnp.full((16,), 0, jnp.int32)])
                er = plsc.load_gather(tabv, [dv, jnp.full((16,), 1, jnp.int32)])
                e = el + er
                e = jnp.where(e >= 0.0, e, 0.2 * e)
                w = jnp.exp(e)
                rows = j * 16 + iota16
                for c in range(3):
                    zc = plsc.load_gather(
                        tabv, [sv, jnp.full((16,), 2 + c, jnp.int32)])
                    plsc.store_scatter(
                        z, [rows, jnp.full((16,), c, jnp.int32)], w * zc)
                plsc.store_scatter(
                    z, [rows, jnp.full((16,), 3, jnp.int32)], w)

        def issue_scatter(b, g):
            pltpu.async_copy(zv[b], acc.at[dstv.at[g]], ss[b], add=True)

        def wait_scatter(b, g):
            pltpu.make_async_copy(zv[b], acc.at[dstv.at[g]], ss[b]).wait()

        @pl.loop(0, _NCH2 - 1, step=2)
        def _pair(g):
            compute(0, g)
            issue_scatter(0, g)
            compute(1, g + 1)
            issue_scatter(1, g + 1)
            wait_scatter(0, g)
            wait_scatter(1, g + 1)

        compute(0, _NCH2 - 1)
        issue_scatter(0, _NCH2 - 1)
        wait_scatter(0, _NCH2 - 1)

        plsc.subcore_barrier()
        pltpu.sync_copy(acc.at[pl.ds(sid * _RPS, _RPS)],
                        out_hbm.at[cid, pl.ds(sid * _RPS, _RPS)])

        @pl.when(sid == _NS - 1)
        def _out_tail():
            pltpu.sync_copy(acc.at[pl.ds(_NS * _RPS, _RTAIL)],
                            out_hbm.at[cid, pl.ds(_NS * _RPS, _RTAIL)])

    return k(src3, dst3, tab, zeros)


def _tc2_body(accp_ref, b1_ref, w2_ref, va_ref, vb_ref, expand_ref, tab_ref):
    acc = accp_ref[0] + accp_ref[1]
    s = acc[:, _H * _F:_H * _F + _H]
    sx = jnp.dot(s, expand_ref[...], preferred_element_type=jnp.float32)
    h = acc[:, 0:_H * _F] / (sx + 1e-9) + b1_ref[...]
    h = jnp.where(h > 0.0, h, jnp.exp(h) - 1.0)
    z2 = jnp.dot(h, w2_ref[...], preferred_element_type=jnp.float32)
    el2 = jnp.dot(h, va_ref[...], preferred_element_type=jnp.float32)
    er2 = jnp.dot(h, vb_ref[...], preferred_element_type=jnp.float32)
    n = h.shape[0]
    tab_ref[...] = jnp.concatenate(
        [el2, er2, z2, jnp.zeros((n, 3), jnp.float32)], axis=1)


def _tc2(accp, b1, W2, va, vb, EXPAND):
    return pl.pallas_call(
        _tc2_body,
        out_shape=jax.ShapeDtypeStruct((_N, 8), jnp.float32),
    )(accp, b1, W2, va, vb, EXPAND)


def _tc3_body(accp_ref, b2_ref, out_ref):
    acc = accp_ref[0] + accp_ref[1]
    sb = jnp.dot(acc[:, 3:4], jnp.ones((1, 16), jnp.float32),
                 preferred_element_type=jnp.float32)
    out_ref[...] = acc[:, 0:3] / (sb[:, 0:3] + 1e-9) + b2_ref[...]


def _tc3(accp, b2):
    return pl.pallas_call(
        _tc3_body,
        out_shape=jax.ShapeDtypeStruct((_N, 3), jnp.float32),
    )(accp, b2)


def kernel(features, edge_index, W1, attn_l1, attn_r1, b1,
           W2, attn_l2, attn_r2, b2):
    src3 = edge_index[0].reshape(_NTILES, _NCH, _CHUNK)
    dst3 = edge_index[1].reshape(_NTILES, _NCH, _CHUNK)
    eye8 = jnp.eye(_H, dtype=jnp.float32)
    AL = (attn_l1[:, :, None] * eye8[:, None, :]).reshape(_H * _F, _H)
    AR = (attn_r1[:, :, None] * eye8[:, None, :]).reshape(_H * _F, _H)
    EXPAND = jnp.kron(eye8, jnp.ones((1, _F), jnp.float32))
    va = (W2 @ attn_l2[0]).reshape(_H * _F, 1)
    vb = (W2 @ attn_r2[0]).reshape(_H * _F, 1)

    zext1, R1 = _tc1(features, W1, AL, AR)
    acc1 = _sc_edge_pass1(src3, dst3, zext1, R1)
    src3b = edge_index[0].reshape(_NTILES, _NCH2, _C2)
    dst3b = edge_index[1].reshape(_NTILES, _NCH2, _C2)
    tab2 = _tc2(acc1, b1.reshape(1, _H * _F), W2, va, vb, EXPAND)
    acc2 = _sc_edge_pass2(src3b, dst3b, tab2)
    out = _tc3(acc2, b2.reshape(1, 3))
    return out.reshape(_N, 1, 3)


# speedup vs baseline: 1.5658x; 1.0120x over previous
import functools

import jax
import jax.numpy as jnp
from jax import lax
from jax.experimental import pallas as pl
from jax.experimental.pallas import tpu as pltpu
from jax.experimental.pallas import tpu_sc as plsc

_N = 10000
_E = 320000
_H = 8
_F = 16
_D1 = 136
_D2 = 16
_NC = 2
_NS = 16
_NTILES = _NC * _NS
_EPT = _E // _NTILES
_CHUNK = 100
_NCH = _EPT // _CHUNK
_RPS = 624
_RTAIL = _N - _NS * _RPS


def _tc1_body(x_ref, w_ref, al_ref, ar_ref, zext_ref, r_ref):
    z = jnp.dot(x_ref[...], w_ref[...], preferred_element_type=jnp.float32)
    el = jnp.dot(z, al_ref[...], preferred_element_type=jnp.float32)
    er = jnp.dot(z, ar_ref[...], preferred_element_type=jnp.float32)
    zext_ref[...] = jnp.concatenate([z, el], axis=1)
    r_ref[...] = jnp.concatenate([er, er], axis=1)


def _tc1(x, W1, AL, AR):
    return pl.pallas_call(
        _tc1_body,
        out_shape=(
            jax.ShapeDtypeStruct((_N, _D1), jnp.float32),
            jax.ShapeDtypeStruct((_N, 16), jnp.float32),
        ),
    )(x, W1, AL, AR)


def _sc_edge_pass1(src3, dst3, zext, rtab):
    mesh = plsc.VectorSubcoreMesh(core_axis_name="c", subcore_axis_name="s")
    zeros = jnp.zeros((_N, _D1), jnp.float32)

    @functools.partial(
        pl.kernel,
        mesh=mesh,
        out_type=jax.ShapeDtypeStruct((_NC, _N, _D1), jnp.float32),
        compiler_params=pltpu.CompilerParams(use_tc_tiling_on_sc=False,
                                             needs_layout_passes=False),
        scratch_types=[
            pltpu.VMEM_SHARED((_N, _D1), jnp.float32),
            pltpu.VMEM((_NCH, _CHUNK), jnp.int32),
            pltpu.VMEM((_CHUNK,), jnp.int32),
            pltpu.VMEM((_CHUNK,), jnp.int32),
            pltpu.VMEM((_CHUNK, 16), jnp.float32),
            pltpu.VMEM((_CHUNK, 16), jnp.float32),
            pltpu.VMEM((_CHUNK, _D1), jnp.float32),
            pltpu.VMEM((_CHUNK, _D1), jnp.float32),
            pltpu.SemaphoreType.DMA,
            pltpu.SemaphoreType.DMA,
            pltpu.SemaphoreType.DMA,
            pltpu.SemaphoreType.DMA,
            pltpu.SemaphoreType.DMA,
            pltpu.SemaphoreType.DMA,
        ],
    )
    def k(src_hbm, dst_hbm, z_hbm, r_hbm, zero_hbm, out_hbm,
          acc, dstv, sv0, sv1, rv0, rv1, zv0, zv1,
          sg0, sg1, ss0, ss1, si0, si1):
        cid = lax.axis_index("c")
        sid = lax.axis_index("s")
        wid = cid * _NS + sid
        sv = (sv0, sv1)
        rv = (rv0, rv1)
        zv = (zv0, zv1)
        sg = (sg0, sg1)
        ss = (ss0, ss1)
        si = (si0, si1)

        pltpu.sync_copy(zero_hbm.at[pl.ds(sid * _RPS, _RPS)],
                        acc.at[pl.ds(sid * _RPS, _RPS)])

        @pl.when(sid == _NS - 1)
        def _zero_tail():
            pltpu.sync_copy(zero_hbm.at[pl.ds(_NS * _RPS, _RTAIL)],
                            acc.at[pl.ds(_NS * _RPS, _RTAIL)])

        pltpu.sync_copy(dst_hbm.at[wid], dstv)
        plsc.subcore_barrier()

        iota16 = lax.iota(jnp.int32, 16)

        def issue_src(b, g):
            pltpu.async_copy(src_hbm.at[wid, g], sv[b], si[b])

        def wait_src(b, g):
            pltpu.make_async_copy(src_hbm.at[wid, g], sv[b], si[b]).wait()

        def issue_gather(b, g):
            pltpu.async_copy(r_hbm.at[dstv.at[g]], rv[b], sg[b])
            pltpu.async_copy(z_hbm.at[sv[b]], zv[b], sg[b])

        def wait_gather(b, g):
            pltpu.make_async_copy(r_hbm.at[dstv.at[g]], rv[b], sg[b]).wait()
            pltpu.make_async_copy(z_hbm.at[sv[b]], zv[b], sg[b]).wait()

        def issue_scatter(b, g):
            pltpu.async_copy(zv[b], acc.at[dstv.at[g]], ss[b], add=True)

        def wait_scatter(b, g):
            pltpu.make_async_copy(zv[b], acc.at[dstv.at[g]], ss[b]).wait()

        def compute(b):
            z = zv[b]
            r = rv[b]

            @plsc.parallel_loop(0, _CHUNK, unroll=2)
            def _msg(ci):
                v = z[ci, pl.ds(120, 16)]
                e = v + r[ci]
                e = jnp.where(e >= 0.0, e, 0.2 * e)
                w16 = jnp.exp(e)
                for h in range(_H):
                    ws = w16[8 + h]
                    z[ci, pl.ds(h * _F, 16)] = ws * z[ci, pl.ds(h * _F, 16)]
                cur = z[ci, pl.ds(120, 16)]
                z[ci, pl.ds(120, 16)] = jnp.where(iota16 < 8, cur, w16)

        pltpu.sync_copy(src_hbm.at[wid, 0], sv0)
        issue_gather(0, 0)
        issue_src(1, 1)

        @pl.loop(0, _NCH, step=2)
        def _pair(g):
            wait_src(1, g + 1)
            issue_gather(1, g + 1)
            wait_gather(0, g)

            @pl.when(g + 2 < _NCH)
            def _s0():
                issue_src(0, g + 2)

            compute(0)
            issue_scatter(0, g)
            wait_gather(1, g + 1)

            @pl.when(g + 3 < _NCH)
            def _s1():
                issue_src(1, g + 3)

            compute(1)
            issue_scatter(1, g + 1)
            wait_scatter(0, g)

            @pl.when(g + 2 < _NCH)
            def _g0():
                wait_src(0, g + 2)
                issue_gather(0, g + 2)

            wait_scatter(1, g + 1)

        plsc.subcore_barrier()
        pltpu.sync_copy(acc.at[pl.ds(sid * _RPS, _RPS)],
                        out_hbm.at[cid, pl.ds(sid * _RPS, _RPS)])

        @pl.when(sid == _NS - 1)
        def _out_tail():
            pltpu.sync_copy(acc.at[pl.ds(_NS * _RPS, _RTAIL)],
                            out_hbm.at[cid, pl.ds(_NS * _RPS, _RTAIL)])

    return k(src3, dst3, zext, rtab, zeros)


_C2 = 80
_NCH2 = _EPT // _C2


def _sc_edge_pass2(src3, dst3, tab):
    mesh = plsc.VectorSubcoreMesh(core_axis_name="c", subcore_axis_name="s")
    zeros = jnp.zeros((_N, _D2), jnp.float32)

    @functools.partial(
        pl.kernel,
        mesh=mesh,
        out_type=jax.ShapeDtypeStruct((_NC, _N, _D2), jnp.float32),
        compiler_params=pltpu.CompilerParams(use_tc_tiling_on_sc=False,
                                             needs_layout_passes=False),
        scratch_types=[
            pltpu.VMEM_SHARED((_N, _D2), jnp.float32),
            pltpu.VMEM((_N, 8), jnp.float32),
            pltpu.VMEM((_NCH2, _C2), jnp.int32),
            pltpu.VMEM((_NCH2, _C2), jnp.int32),
            pltpu.VMEM((_C2, _D2), jnp.float32),
            pltpu.VMEM((_C2, _D2), jnp.float32),
            pltpu.SemaphoreType.DMA,
            pltpu.SemaphoreType.DMA,
        ],
    )
    def k(src_hbm, dst_hbm, tab_hbm, zero_hbm, out_hbm,
          acc, tabv, srcv, dstv, zv0, zv1, ss0, ss1):
        cid = lax.axis_index("c")
        sid = lax.axis_index("s")
        wid = cid * _NS + sid
        zv = (zv0, zv1)
        ss = (ss0, ss1)

        pltpu.sync_copy(zero_hbm.at[pl.ds(sid * _RPS, _RPS)],
                        acc.at[pl.ds(sid * _RPS, _RPS)])

        @pl.when(sid == _NS - 1)
        def _zero_tail():
            pltpu.sync_copy(zero_hbm.at[pl.ds(_NS * _RPS, _RTAIL)],
                            acc.at[pl.ds(_NS * _RPS, _RTAIL)])

        pltpu.sync_copy(tab_hbm, tabv)
        pltpu.sync_copy(src_hbm.at[wid], srcv)
        pltpu.sync_copy(dst_hbm.at[wid], dstv)
        plsc.subcore_barrier()

        iota16 = lax.iota(jnp.int32, 16)

        def compute(b, g):
            z = zv[b]

            @plsc.parallel_loop(0, _C2 // 16, unroll=5)
            def _grp(j):
                sv = srcv[g, pl.ds(j * 16, 16)]
                dv = dstv[g, pl.ds(j * 16, 16)]
                el = plsc.load_gather(tabv, [sv, jnp.full((16,), 0, jnp.int32)])
                er = plsc.load_gather(tabv, [dv, jnp.full((16,), 1, jnp.int32)])
                e = el + er
                e = jnp.where(e >= 0.0, e, 0.2 * e)
                w = jnp.exp(e)
                rows = j * 16 + iota16
                for c in range(3):
                    zc = plsc.load_gather(
                        tabv, [sv, jnp.full((16,), 2 + c, jnp.int32)])
                    plsc.store_scatter(
                        z, [rows, jnp.full((16,), c, jnp.int32)], w * zc)
                plsc.store_scatter(
                    z, [rows, jnp.full((16,), 3, jnp.int32)], w)

        def issue_scatter(b, g):
            pltpu.async_copy(zv[b], acc.at[dstv.at[g]], ss[b], add=True)

        def wait_scatter(b, g):
            pltpu.make_async_copy(zv[b], acc.at[dstv.at[g]], ss[b]).wait()

        @pl.loop(0, _NCH2 - 1, step=2)
        def _pair(g):
            compute(0, g)
            issue_scatter(0, g)
            compute(1, g + 1)
            issue_scatter(1, g + 1)
            wait_scatter(0, g)
            wait_scatter(1, g + 1)

        compute(0, _NCH2 - 1)
        issue_scatter(0, _NCH2 - 1)
        wait_scatter(0, _NCH2 - 1)

        plsc.subcore_barrier()
        pltpu.sync_copy(acc.at[pl.ds(sid * _RPS, _RPS)],
                        out_hbm.at[cid, pl.ds(sid * _RPS, _RPS)])

        @pl.when(sid == _NS - 1)
        def _out_tail():
            pltpu.sync_copy(acc.at[pl.ds(_NS * _RPS, _RTAIL)],
                            out_hbm.at[cid, pl.ds(_NS * _RPS, _RTAIL)])

    return k(src3, dst3, tab, zeros)


def _tc2_body(accp_ref, b1_ref, w2_ref, va_ref, vb_ref, expand_ref, tab_ref):
    acc = accp_ref[0] + accp_ref[1]
    s = acc[:, _H * _F:_H * _F + _H]
    sx = jnp.dot(s, expand_ref[...], preferred_element_type=jnp.float32)
    h = acc[:, 0:_H * _F] / (sx + 1e-9) + b1_ref[...]
    h = jnp.where(h > 0.0, h, jnp.exp(h) - 1.0)
    z2 = jnp.dot(h, w2_ref[...], preferred_element_type=jnp.float32)
    el2 = jnp.dot(h, va_ref[...], preferred_element_type=jnp.float32)
    er2 = jnp.dot(h, vb_ref[...], preferred_element_type=jnp.float32)
    n = h.shape[0]
    tab_ref[...] = jnp.concatenate(
        [el2, er2, z2, jnp.zeros((n, 3), jnp.float32)], axis=1)


def _tc2(accp, b1, W2, va, vb, EXPAND):
    return pl.pallas_call(
        _tc2_body,
        out_shape=jax.ShapeDtypeStruct((_N, 8), jnp.float32),
    )(accp, b1, W2, va, vb, EXPAND)


def _tc3_body(accp_ref, b2_ref, out_ref):
    acc = accp_ref[0] + accp_ref[1]
    sb = jnp.dot(acc[:, 3:4], jnp.ones((1, 16), jnp.float32),
                 preferred_element_type=jnp.float32)
    out_ref[...] = acc[:, 0:3] / (sb[:, 0:3] + 1e-9) + b2_ref[...]


def _tc3(accp, b2):
    return pl.pallas_call(
        _tc3_body,
        out_shape=jax.ShapeDtypeStruct((_N, 3), jnp.float32),
    )(accp, b2)


def kernel(features, edge_index, W1, attn_l1, attn_r1, b1,
           W2, attn_l2, attn_r2, b2):
    src3 = edge_index[0].reshape(_NTILES, _NCH, _CHUNK)
    dst3 = edge_index[1].reshape(_NTILES, _NCH, _CHUNK)
    eye8 = jnp.eye(_H, dtype=jnp.float32)
    AL = (attn_l1[:, :, None] * eye8[:, None, :]).reshape(_H * _F, _H)
    AR = (attn_r1[:, :, None] * eye8[:, None, :]).reshape(_H * _F, _H)
    EXPAND = jnp.kron(eye8, jnp.ones((1, _F), jnp.float32))
    va = (W2 @ attn_l2[0]).reshape(_H * _F, 1)
    vb = (W2 @ attn_r2[0]).reshape(_H * _F, 1)

    zext1, R1 = _tc1(features, W1, AL, AR)
    acc1 = _sc_edge_pass1(src3, dst3, zext1, R1)
    src3b = edge_index[0].reshape(_NTILES, _NCH2, _C2)
    dst3b = edge_index[1].reshape(_NTILES, _NCH2, _C2)
    tab2 = _tc2(acc1, b1.reshape(1, _H * _F), W2, va, vb, EXPAND)
    acc2 = _sc_edge_pass2(src3b, dst3b, tab2)
    out = _tc3(acc2, b2.reshape(1, 3))
    return out.reshape(_N, 1, 3)
